# baseline (speedup 1.0000x reference)
import functools

import jax
import jax.numpy as jnp
from jax import lax
from jax.experimental import pallas as pl
from jax.experimental.pallas import tpu as pltpu
from jax.experimental.pallas import tpu_sc as plsc

TOKENS = 32768
D_MODEL = 4096
N_MODELS = 64
TOPK = 8
BLOCK_T = 1024

N_WORKERS = 32
T_PER_W = TOKENS // N_WORKERS
CHUNK = 256
LANES = 16


def _dense_body(x_ref, w_ref, nz_ref, o_ref):
    acc = jnp.dot(
        x_ref[:].astype(jnp.bfloat16),
        w_ref[:],
        preferred_element_type=jnp.float32,
    )
    hg = acc[:, :N_MODELS]
    sp = acc[:, N_MODELS:]
    o_ref[:] = hg + nz_ref[:] * jnp.logaddexp(sp, 0.0)


def _dense(x, w_cat, noise):
    grid = (TOKENS // BLOCK_T,)
    return pl.pallas_call(
        _dense_body,
        grid=grid,
        in_specs=[
            pl.BlockSpec((BLOCK_T, D_MODEL), lambda i: (i, 0)),
            pl.BlockSpec((D_MODEL, 2 * N_MODELS), lambda i: (0, 0)),
            pl.BlockSpec((BLOCK_T, N_MODELS), lambda i: (i, 0)),
        ],
        out_specs=pl.BlockSpec((BLOCK_T, N_MODELS), lambda i: (i, 0)),
        out_shape=jax.ShapeDtypeStruct((TOKENS, N_MODELS), jnp.float32),
        compiler_params=pltpu.CompilerParams(
            dimension_semantics=("arbitrary",),
        ),
    )(x, w_cat, noise)


def _expert_cols(lane, e):
    c = lane + e
    return jnp.where(c >= N_MODELS, c - N_MODELS, c)


def _sc_route_body(h_hbm, g_hbm, in_v, out_v):
    wid = lax.axis_index("s") * 2 + lax.axis_index("c")
    base = wid * (T_PER_W * N_MODELS)
    lane = lax.iota(jnp.int32, LANES)
    lane64 = lane * N_MODELS
    neg_inf = jnp.full((LANES,), -jnp.inf, jnp.float32)
    chunk_words = CHUNK * N_MODELS

    def chunk_body(ci, carry):
        off = base + ci * chunk_words
        pltpu.sync_copy(h_hbm.at[pl.ds(off, chunk_words)], in_v)

        def group_body(g, carry2):
            goff = lane64 + g * (LANES * N_MODELS)
            t = [neg_inf] * TOPK
            for e in range(N_MODELS):
                idx = goff + _expert_cols(lane, e)
                x = plsc.load_gather(in_v, [idx])
                for j in range(TOPK):
                    hi = jnp.maximum(t[j], x)
                    x = jnp.minimum(t[j], x)
                    t[j] = hi
            row_max, kth = t[0], t[TOPK - 1]
            s = jnp.zeros((LANES,), jnp.float32)
            for e in range(N_MODELS):
                idx = goff + _expert_cols(lane, e)
                x = plsc.load_gather(in_v, [idx])
                ex = jnp.where(x >= kth, jnp.exp(x - row_max), 0.0)
                s = s + ex
                plsc.store_scatter(out_v, [idx], ex)
            inv = 1.0 / s
            for e in range(N_MODELS):
                idx = goff + _expert_cols(lane, e)
                ex = plsc.load_gather(out_v, [idx])
                plsc.store_scatter(out_v, [idx], ex * inv)
            return carry2

        lax.fori_loop(0, CHUNK // LANES, group_body, 0)
        pltpu.sync_copy(out_v, g_hbm.at[pl.ds(off, chunk_words)])
        return carry

    lax.fori_loop(0, T_PER_W // CHUNK, chunk_body, 0)


_sc_route = functools.partial(
    pl.kernel,
    mesh=plsc.VectorSubcoreMesh(core_axis_name="c", subcore_axis_name="s"),
    out_type=jax.ShapeDtypeStruct((TOKENS * N_MODELS,), jnp.float32),
    scratch_types=[
        pltpu.VMEM((CHUNK * N_MODELS,), jnp.float32),
        pltpu.VMEM((CHUNK * N_MODELS,), jnp.float32),
    ],
    compiler_params=pltpu.CompilerParams(needs_layout_passes=False),
)(_sc_route_body)


def kernel(noise_key, x, W_g, W_noise):
    x2 = x if x.ndim == 2 else x.reshape((x.shape[0], -1))
    noise = jax.random.normal(noise_key, shape=(x2.shape[0], N_MODELS))
    w_cat = jnp.concatenate([W_g, W_noise], axis=1).astype(jnp.bfloat16)
    h = _dense(x2, w_cat, noise)
    g = _sc_route(h.reshape((TOKENS * N_MODELS,)))
    return g.reshape((TOKENS, N_MODELS))
